# baseline (speedup 1.0000x reference)
import functools

import jax
import jax.numpy as jnp
from jax import lax
from jax.experimental import pallas as pl
from jax.experimental.pallas import tpu as pltpu
from jax.experimental.pallas import tpu_sc as plsc

N = 10000
NPAD = 10240
E = 320000
CHUNK = 128
NCORES = 2
NSUB = 16
NTILES = NCORES * NSUB
NCHUNKS = 2560
EPAD = NCHUNKS * CHUNK
ROWS_PER_TILE = NPAD // NSUB
SIGMA = 0.5
BLK = 1000
RING = 8
NBUF = 8
LOOK = 4


def _spmm_tile_program(s_hbm, pk_hbm, w_hbm, out_hbm,
                       pk_v, w_ring, src_ring, dst_ring, bufs, acc,
                       semg, sems, semw, *, dh, kch, row0, c, s):
    pltpu.sync_copy(pk_hbm.at[pl.ds(row0, kch)], pk_v)

    def zrow(i, carry):
        for k in range(dh // 16):
            bufs[0][i, pl.ds(k * 16, 16)] = jnp.zeros((16,), jnp.float32)
        return carry
    lax.fori_loop(0, CHUNK, zrow, 0)
    for k in range(ROWS_PER_TILE // CHUNK):
        pltpu.sync_copy(bufs[0],
                        acc.at[pl.ds(s * ROWS_PER_TILE + k * CHUNK, CHUNK)])
    plsc.subcore_barrier()

    def unpack(jc):
        slot = jc & (RING - 1)
        jcc = jnp.minimum(jc, kch - 1)

        def group(g, carry):
            sl = pl.ds(g * 16, 16)
            pk = pk_v[jcc, sl]
            src_ring[slot, sl] = pk & 0xFFFF
            dst_ring[slot, sl] = pk >> 16
            return carry
        lax.fori_loop(0, CHUNK // 16, group, 0)

    def issue_gather(jc, b):
        slot = jc & (RING - 1)
        pltpu.async_copy(s_hbm.at[src_ring.at[slot]], bufs[b], semg[b])

    def issue_w(jc):
        slot = jc & (RING - 1)
        jcc = jnp.minimum(jc, kch - 1)
        pltpu.async_copy(w_hbm.at[row0 + jcc], w_ring.at[slot], semw)

    for b in range(LOOK):
        unpack(jnp.int32(b))
        issue_gather(jnp.int32(b), b)
        issue_w(jnp.int32(b))

    def outer(t, carry):
        for b0 in range(NBUF):
            jj = NBUF * t + b0
            bj = b0
            bg = (b0 + LOOK) % NBUF

            @pl.when(jj >= LOOK)
            def _wait_prev_scatter():
                pltpu.make_async_copy(
                    bufs[bg], acc.at[dst_ring.at[(jj - LOOK) & (RING - 1)]],
                    sems[bg]).wait()

            unpack(jj + LOOK)
            issue_gather(jj + LOOK, bg)
            issue_w(jj + LOOK)

            pltpu.make_async_copy(
                s_hbm.at[src_ring.at[jj & (RING - 1)]], bufs[bj],
                semg[bj]).wait()
            pltpu.make_async_copy(
                w_hbm.at[row0], w_ring.at[jj & (RING - 1)], semw).wait()

            def group(g, gcarry):
                wv = w_ring[jj & (RING - 1), pl.ds(g * 16, 16)]
                base = g * 16
                for i in range(16):
                    w = wv[i]
                    for k in range(dh // 16):
                        sl = pl.ds(k * 16, 16)
                        bufs[bj][base + i, sl] = bufs[bj][base + i, sl] * w
                return gcarry
            lax.fori_loop(0, CHUNK // 16, group, 0)

            pltpu.async_copy(bufs[bj], acc.at[dst_ring.at[jj & (RING - 1)]],
                             sems[bj], add=True)
        return carry
    lax.fori_loop(0, kch // NBUF, outer, 0)

    for b in range(LOOK):
        pltpu.make_async_copy(
            s_hbm.at[src_ring.at[(kch + b) % RING]], bufs[b], semg[b]).wait()
        pltpu.make_async_copy(
            w_hbm.at[row0], w_ring.at[(kch + b) % RING], semw).wait()
    for jj in range(kch - LOOK, kch):
        pltpu.make_async_copy(
            bufs[jj % NBUF], acc.at[dst_ring.at[jj % RING]],
            sems[jj % NBUF]).wait()
    plsc.subcore_barrier()

    for k in range(ROWS_PER_TILE // CHUNK):
        r0 = s * ROWS_PER_TILE + k * CHUNK
        pltpu.sync_copy(acc.at[pl.ds(r0, CHUNK)],
                        out_hbm.at[pl.ds(c * NPAD + r0, CHUNK)])


@functools.cache
def _make_spmm(dh, edge_split):
    kch = NCHUNKS // NTILES if edge_split else NCHUNKS // NSUB
    mesh = plsc.VectorSubcoreMesh(core_axis_name="c", subcore_axis_name="s")

    @functools.partial(
        pl.kernel,
        mesh=mesh,
        out_type=jax.ShapeDtypeStruct((NCORES * NPAD, dh), jnp.float32),
        scratch_types=[
            pltpu.VMEM((kch, CHUNK), jnp.int32),
            pltpu.VMEM((RING, CHUNK), jnp.float32),
            pltpu.VMEM((RING, CHUNK), jnp.int32),
            pltpu.VMEM((RING, CHUNK), jnp.int32),
        ] + [pltpu.VMEM((CHUNK, dh), jnp.float32) for _ in range(NBUF)]
          + [pltpu.VMEM_SHARED((NPAD, dh), jnp.float32)]
          + [pltpu.SemaphoreType.DMA for _ in range(2 * NBUF + 1)],
        compiler_params=pltpu.CompilerParams(use_tc_tiling_on_sc=False),
    )
    def spmm(s_hbm, pk_hbm, w_hbm, out_hbm,
             pk_v, w_ring, src_ring, dst_ring, *rest):
        bufs = rest[:NBUF]
        acc = rest[NBUF]
        semg = rest[NBUF + 1:2 * NBUF + 1]
        sems = rest[2 * NBUF + 1:3 * NBUF + 1]
        semw = rest[3 * NBUF + 1]
        c = lax.axis_index("c")
        s = lax.axis_index("s")
        if edge_split:
            s_view = s_hbm
            row0 = (c * NSUB + s) * kch
        else:
            s_view = s_hbm.at[c]
            row0 = s * kch
        _spmm_tile_program(
            s_view, pk_hbm, w_hbm, out_hbm, pk_v, w_ring, src_ring, dst_ring,
            bufs, acc, semg, sems, semw,
            dh=dh, kch=kch, row0=row0, c=c, s=s)

    return spmm


def _spmm_cols(s2, pk, w):
    dh = s2.shape[-1]
    out = _make_spmm(dh, False)(s2, pk, w)
    return out.reshape(NCORES, NPAD, dh)


def _spmm_edges(s, pk, w):
    d = s.shape[-1]
    out = _make_spmm(d, True)(s, pk, w)
    return out.reshape(NCORES, NPAD, d)


def _mm_split_body(x_ref, w_ref, o_ref):
    r = jnp.dot(x_ref[...], w_ref[...], preferred_element_type=jnp.float32)
    dh = r.shape[-1] // 2
    o_ref[0] = r[:, :dh]
    o_ref[1] = r[:, dh:]


def _mm_split(x, w):
    din, dout = w.shape
    dh = dout // 2
    return pl.pallas_call(
        _mm_split_body,
        grid=(N // BLK,),
        in_specs=[
            pl.BlockSpec((BLK, din), lambda i: (i, 0)),
            pl.BlockSpec((din, dout), lambda i: (0, 0)),
        ],
        out_specs=pl.BlockSpec((NCORES, BLK, dh), lambda i: (0, i, 0)),
        out_shape=jax.ShapeDtypeStruct((NCORES, N, dh), jnp.float32),
    )(x, w)


def _fuse_body(split_out, p_ref, t_ref, w_ref, o_ref):
    din2 = p_ref.shape[-1]
    h = jnp.concatenate(
        [jnp.maximum(p_ref[0], 0.0), jnp.maximum(p_ref[1], 0.0)], axis=1)
    h = (1.0 - SIGMA) * h + SIGMA * t_ref[...]
    r = jnp.dot(h, w_ref[...], preferred_element_type=jnp.float32)
    if split_out:
        dh = r.shape[-1] // 2
        o_ref[0] = r[:, :dh]
        o_ref[1] = r[:, dh:]
    else:
        o_ref[...] = r


def _fuse(p, t, w, split_out):
    din, dout = w.shape
    din2 = din // 2
    if split_out:
        dh = dout // 2
        out_specs = pl.BlockSpec((NCORES, BLK, dh), lambda i: (0, i, 0))
        out_shape = jax.ShapeDtypeStruct((NCORES, N, dh), jnp.float32)
    else:
        out_specs = pl.BlockSpec((BLK, dout), lambda i: (i, 0))
        out_shape = jax.ShapeDtypeStruct((N, dout), jnp.float32)
    return pl.pallas_call(
        functools.partial(_fuse_body, split_out),
        grid=(N // BLK,),
        in_specs=[
            pl.BlockSpec((NCORES, BLK, din2), lambda i: (0, i, 0)),
            pl.BlockSpec((BLK, din), lambda i: (i, 0)),
            pl.BlockSpec((din, dout), lambda i: (0, 0)),
        ],
        out_specs=out_specs,
        out_shape=out_shape,
    )(p, t, w)


def _softmax_body(p_ref, o_ref):
    h = p_ref[0] + p_ref[1]
    m = jnp.max(h, axis=1, keepdims=True)
    e = jnp.exp(h - m)
    o_ref[...] = e / jnp.sum(e, axis=1, keepdims=True)


def _softmax(p):
    d = p.shape[-1]
    return pl.pallas_call(
        _softmax_body,
        grid=(N // BLK,),
        in_specs=[pl.BlockSpec((NCORES, BLK, d), lambda i: (0, i, 0))],
        out_specs=pl.BlockSpec((BLK, d), lambda i: (i, 0)),
        out_shape=jax.ShapeDtypeStruct((N, d), jnp.float32),
    )(p)


def kernel(x, edge_index, edge_weight, tra1, tra2, tra3, z,
           W1, W2, W3, W4, W5):
    src = edge_index[1].astype(jnp.int32)
    dst = edge_index[0].astype(jnp.int32)
    order = jnp.argsort(src)
    src = src[order]
    dst = dst[order]
    edge_weight = edge_weight[order]
    pad = EPAD - E
    packed = jnp.pad(src | (dst << 16), (0, pad)).reshape(NCHUNKS, CHUNK)
    wp = jnp.pad(edge_weight, (0, pad)).reshape(NCHUNKS, CHUNK)

    s1 = _mm_split(x, W1)
    p1 = _spmm_cols(s1, packed, wp)
    s2 = _fuse(p1, tra1, W2, True)
    p2 = _spmm_cols(s2, packed, wp)
    s3 = _fuse(p2, tra2, W3, True)
    p3 = _spmm_cols(s3, packed, wp)
    s4 = _fuse(p3, tra3, W4, True)
    p4 = _spmm_cols(s4, packed, wp)
    s5 = _fuse(p4, z, W5, False)
    p5 = _spmm_edges(s5, packed, wp)
    return _softmax(p5)


# speedup vs baseline: 1.7388x; 1.7388x over previous
import functools

import jax
import jax.numpy as jnp
from jax import lax
from jax.experimental import pallas as pl
from jax.experimental.pallas import tpu as pltpu
from jax.experimental.pallas import tpu_sc as plsc

N = 10000
NPAD = 10240
E = 320000
CHUNK = 128
NCORES = 2
NSUB = 16
NTILES = NCORES * NSUB
NCHUNKS = 2560
EPAD = NCHUNKS * CHUNK
ROWS_PER_TILE = NPAD // NSUB
SIGMA = 0.5
BLK = 1000
RING = 8
NBUF = 4


def _spmm_tile_program(s_hbm, pk_hbm, w_hbm, out_hbm,
                       pk_v, w_v, src_ring, dst_ring, bufs, acc,
                       semg, sems, *, dh, kch, row0, c, s):
    pltpu.sync_copy(pk_hbm.at[pl.ds(row0, kch)], pk_v)
    pltpu.sync_copy(w_hbm.at[pl.ds(row0, kch)], w_v)

    def zrow(i, carry):
        for k in range(dh // 16):
            bufs[0][i, pl.ds(k * 16, 16)] = jnp.zeros((16,), jnp.float32)
        return carry
    lax.fori_loop(0, CHUNK, zrow, 0)
    for k in range(ROWS_PER_TILE // CHUNK):
        pltpu.sync_copy(bufs[0],
                        acc.at[pl.ds(s * ROWS_PER_TILE + k * CHUNK, CHUNK)])
    plsc.subcore_barrier()

    def unpack(jc):
        slot = jc & (RING - 1)
        jcc = jnp.minimum(jc, kch - 1)

        def group(g, carry):
            sl = pl.ds(g * 16, 16)
            pk = pk_v[jcc, sl]
            src_ring[slot, sl] = pk & 0xFFFF
            dst_ring[slot, sl] = pk >> 16
            return carry
        lax.fori_loop(0, CHUNK // 16, group, 0)

    def issue_gather(jc, b):
        slot = jc & (RING - 1)
        pltpu.async_copy(s_hbm.at[src_ring.at[slot]], bufs[b], semg[b])

    for b in range(2):
        unpack(jnp.int32(b))
        issue_gather(jnp.int32(b), b)

    def outer(t, carry):
        for b0 in range(NBUF):
            jj = NBUF * t + b0
            bj = b0
            bg = (b0 + 2) % NBUF

            @pl.when(jj >= 2)
            def _wait_prev_scatter():
                pltpu.make_async_copy(
                    bufs[bg], acc.at[dst_ring.at[(jj - 2) & (RING - 1)]],
                    sems[bg]).wait()

            unpack(jj + 2)
            issue_gather(jj + 2, bg)

            pltpu.make_async_copy(
                s_hbm.at[src_ring.at[jj & (RING - 1)]], bufs[bj],
                semg[bj]).wait()

            def group(g, gcarry):
                wv = w_v[jj, pl.ds(g * 16, 16)]
                base = g * 16
                ws = [wv[i] for i in range(16)]
                for k in range(dh // 16):
                    sl = pl.ds(k * 16, 16)
                    for i in range(16):
                        bufs[bj][base + i, sl] = bufs[bj][base + i, sl] * ws[i]
                return gcarry
            lax.fori_loop(0, CHUNK // 16, group, 0)

            pltpu.async_copy(bufs[bj], acc.at[dst_ring.at[jj & (RING - 1)]],
                             sems[bj], add=True)
        return carry
    lax.fori_loop(0, kch // NBUF, outer, 0)

    for b in range(2):
        pltpu.make_async_copy(
            s_hbm.at[src_ring.at[(kch + b) % RING]], bufs[b], semg[b]).wait()
    for jj in (kch - 2, kch - 1):
        pltpu.make_async_copy(
            bufs[jj % NBUF], acc.at[dst_ring.at[jj % RING]],
            sems[jj % NBUF]).wait()
    plsc.subcore_barrier()

    for k in range(ROWS_PER_TILE // CHUNK):
        r0 = s * ROWS_PER_TILE + k * CHUNK
        pltpu.sync_copy(acc.at[pl.ds(r0, CHUNK)],
                        out_hbm.at[pl.ds(c * NPAD + r0, CHUNK)])


@functools.cache
def _make_spmm(dh, edge_split):
    kch = NCHUNKS // NTILES if edge_split else NCHUNKS // NSUB
    mesh = plsc.VectorSubcoreMesh(core_axis_name="c", subcore_axis_name="s")

    @functools.partial(
        pl.kernel,
        mesh=mesh,
        out_type=jax.ShapeDtypeStruct((NCORES * NPAD, dh), jnp.float32),
        scratch_types=[
            pltpu.VMEM((kch, CHUNK), jnp.int32),
            pltpu.VMEM((kch, CHUNK), jnp.float32),
            pltpu.VMEM((RING, CHUNK), jnp.int32),
            pltpu.VMEM((RING, CHUNK), jnp.int32),
        ] + [pltpu.VMEM((CHUNK, dh), jnp.float32) for _ in range(NBUF)]
          + [pltpu.VMEM_SHARED((NPAD, dh), jnp.float32)]
          + [pltpu.SemaphoreType.DMA for _ in range(2 * NBUF)],
        compiler_params=pltpu.CompilerParams(use_tc_tiling_on_sc=False),
    )
    def spmm(s_hbm, pk_hbm, w_hbm, out_hbm,
             pk_v, w_v, src_ring, dst_ring,
             b0, b1, b2, b3, acc, g0, g1, g2, g3, s0, s1, s2, s3):
        c = lax.axis_index("c")
        s = lax.axis_index("s")
        if edge_split:
            s_view = s_hbm
            row0 = (c * NSUB + s) * kch
        else:
            s_view = s_hbm.at[c]
            row0 = s * kch
        _spmm_tile_program(
            s_view, pk_hbm, w_hbm, out_hbm, pk_v, w_v, src_ring, dst_ring,
            (b0, b1, b2, b3), acc, (g0, g1, g2, g3), (s0, s1, s2, s3),
            dh=dh, kch=kch, row0=row0, c=c, s=s)

    return spmm


def _spmm_cols(s2, pk, w):
    dh = s2.shape[-1]
    out = _make_spmm(dh, False)(s2, pk, w)
    return out.reshape(NCORES, NPAD, dh)


def _spmm_edges(s, pk, w):
    d = s.shape[-1]
    out = _make_spmm(d, True)(s, pk, w)
    return out.reshape(NCORES, NPAD, d)


def _mm_split_body(x_ref, w_ref, o_ref):
    r = jnp.dot(x_ref[...], w_ref[...], preferred_element_type=jnp.float32)
    dh = r.shape[-1] // 2
    o_ref[0] = r[:, :dh]
    o_ref[1] = r[:, dh:]


def _mm_split(x, w):
    din, dout = w.shape
    dh = dout // 2
    return pl.pallas_call(
        _mm_split_body,
        grid=(N // BLK,),
        in_specs=[
            pl.BlockSpec((BLK, din), lambda i: (i, 0)),
            pl.BlockSpec((din, dout), lambda i: (0, 0)),
        ],
        out_specs=pl.BlockSpec((NCORES, BLK, dh), lambda i: (0, i, 0)),
        out_shape=jax.ShapeDtypeStruct((NCORES, N, dh), jnp.float32),
    )(x, w)


def _fuse_body(split_out, p_ref, t_ref, w_ref, o_ref):
    din2 = p_ref.shape[-1]
    h = jnp.concatenate(
        [jnp.maximum(p_ref[0], 0.0), jnp.maximum(p_ref[1], 0.0)], axis=1)
    h = (1.0 - SIGMA) * h + SIGMA * t_ref[...]
    r = jnp.dot(h, w_ref[...], preferred_element_type=jnp.float32)
    if split_out:
        dh = r.shape[-1] // 2
        o_ref[0] = r[:, :dh]
        o_ref[1] = r[:, dh:]
    else:
        o_ref[...] = r


def _fuse(p, t, w, split_out):
    din, dout = w.shape
    din2 = din // 2
    if split_out:
        dh = dout // 2
        out_specs = pl.BlockSpec((NCORES, BLK, dh), lambda i: (0, i, 0))
        out_shape = jax.ShapeDtypeStruct((NCORES, N, dh), jnp.float32)
    else:
        out_specs = pl.BlockSpec((BLK, dout), lambda i: (i, 0))
        out_shape = jax.ShapeDtypeStruct((N, dout), jnp.float32)
    return pl.pallas_call(
        functools.partial(_fuse_body, split_out),
        grid=(N // BLK,),
        in_specs=[
            pl.BlockSpec((NCORES, BLK, din2), lambda i: (0, i, 0)),
            pl.BlockSpec((BLK, din), lambda i: (i, 0)),
            pl.BlockSpec((din, dout), lambda i: (0, 0)),
        ],
        out_specs=out_specs,
        out_shape=out_shape,
    )(p, t, w)


def _softmax_body(p_ref, o_ref):
    h = p_ref[0] + p_ref[1]
    m = jnp.max(h, axis=1, keepdims=True)
    e = jnp.exp(h - m)
    o_ref[...] = e / jnp.sum(e, axis=1, keepdims=True)


def _softmax(p):
    d = p.shape[-1]
    return pl.pallas_call(
        _softmax_body,
        grid=(N // BLK,),
        in_specs=[pl.BlockSpec((NCORES, BLK, d), lambda i: (0, i, 0))],
        out_specs=pl.BlockSpec((BLK, d), lambda i: (i, 0)),
        out_shape=jax.ShapeDtypeStruct((N, d), jnp.float32),
    )(p)


def kernel(x, edge_index, edge_weight, tra1, tra2, tra3, z,
           W1, W2, W3, W4, W5):
    src = edge_index[1].astype(jnp.int32)
    dst = edge_index[0].astype(jnp.int32)
    pad = EPAD - E
    packed = jnp.pad(src | (dst << 16), (0, pad)).reshape(NCHUNKS, CHUNK)
    wp = jnp.pad(edge_weight, (0, pad)).reshape(NCHUNKS, CHUNK)

    s1 = _mm_split(x, W1)
    p1 = _spmm_cols(s1, packed, wp)
    s2 = _fuse(p1, tra1, W2, True)
    p2 = _spmm_cols(s2, packed, wp)
    s3 = _fuse(p2, tra2, W3, True)
    p3 = _spmm_cols(s3, packed, wp)
    s4 = _fuse(p3, tra3, W4, True)
    p4 = _spmm_cols(s4, packed, wp)
    s5 = _fuse(p4, z, W5, False)
    p5 = _spmm_edges(s5, packed, wp)
    return _softmax(p5)
